# baseline (speedup 1.0000x reference)
import functools

import jax
import jax.numpy as jnp
from jax import lax
from jax.experimental import pallas as pl
from jax.experimental.pallas import tpu as pltpu
from jax.experimental.pallas import tpu_sc as plsc

_N_NODES = 10000
_D = 128
_N_EDGES = 320000
_NC = 2
_NS = 16
_EPW = _N_EDGES // (_NC * _NS)
_CH = 40
_NCHUNK = _EPW // _CH
_NP = 10240
_RPT = _NP // _NS


_NBUF = 5
_NOUT = _NCHUNK // _NBUF


def _sc_body(j_hbm, ea_hbm, out_hbm, idx_v, rows_v, agg_sh, isem, rsem):
    c = lax.axis_index("c")
    s = lax.axis_index("s")
    zv = jnp.zeros((16,), jnp.float32)
    for r in range(_CH):
        for c16 in range(_D // 16):
            rows_v[0, r, pl.ds(c16 * 16, 16)] = zv
    for k in range(_RPT // _CH):
        pltpu.async_copy(rows_v.at[0],
                         agg_sh.at[pl.ds(s * _RPT + k * _CH, _CH)],
                         isem.at[0])
    for k in range(_RPT // _CH):
        pltpu.make_async_copy(rows_v.at[0],
                              agg_sh.at[pl.ds(s * _RPT + k * _CH, _CH)],
                              isem.at[0]).wait()
    plsc.subcore_barrier()
    ebase0 = (c * _NS + s) * _EPW
    jbase0 = ebase0

    for b in range(_NBUF):
        off = b * _CH
        pltpu.async_copy(j_hbm.at[pl.ds(jbase0 + off, _CH)], idx_v.at[b],
                         isem.at[b])
        pltpu.async_copy(ea_hbm.at[pl.ds(ebase0 + off, _CH)], rows_v.at[b],
                         rsem.at[b])

    def outer(o, carry):
        for b in range(_NBUF):
            off = (o * _NBUF + b) * _CH
            pltpu.make_async_copy(j_hbm.at[pl.ds(jbase0 + off, _CH)],
                                  idx_v.at[b], isem.at[b]).wait()
            pltpu.make_async_copy(ea_hbm.at[pl.ds(ebase0 + off, _CH)],
                                  rows_v.at[b], rsem.at[b]).wait()
            pltpu.sync_copy(rows_v.at[b], agg_sh.at[idx_v.at[b]], add=True)

            @pl.when(o < _NOUT - 1)
            def _():
                noff = off + _NBUF * _CH
                pltpu.async_copy(j_hbm.at[pl.ds(jbase0 + noff, _CH)],
                                 idx_v.at[b], isem.at[b])
                pltpu.async_copy(ea_hbm.at[pl.ds(ebase0 + noff, _CH)],
                                 rows_v.at[b], rsem.at[b])
        return carry

    lax.fori_loop(0, _NOUT, outer, 0)
    plsc.subcore_barrier()
    dst = out_hbm.at[pl.ds(c * _NP + s * _RPT, _RPT)]
    pltpu.sync_copy(agg_sh.at[pl.ds(s * _RPT, _RPT)], dst)


@functools.cache
def _sc_scatter():
    return pl.kernel(
        _sc_body,
        out_type=jax.ShapeDtypeStruct((_NC * _NP, _D), jnp.float32),
        mesh=plsc.VectorSubcoreMesh(core_axis_name="c", subcore_axis_name="s",
                                    num_cores=_NC, num_subcores=_NS),
        scratch_types=[
            pltpu.VMEM((_NBUF, _CH), jnp.int32),
            pltpu.VMEM((_NBUF, _CH, _D), jnp.float32),
            pltpu.VMEM_SHARED((_NP, _D), jnp.float32),
            pltpu.SemaphoreType.DMA((_NBUF,)),
            pltpu.SemaphoreType.DMA((_NBUF,)),
        ],
    )


_BLK = 1280
_NBLK = _NP // _BLK


def _mlp1_body(x_ref, w1a_ref, b1_ref, hx_ref):
    hx_ref[...] = (jnp.dot(x_ref[...], w1a_ref[...],
                           preferred_element_type=jnp.float32) + b1_ref[...])


def _tc_mlp1(x, w1a, b1):
    return pl.pallas_call(
        _mlp1_body,
        grid=(_NBLK,),
        in_specs=[
            pl.BlockSpec((_BLK, _D), lambda i: (i, 0)),
            pl.BlockSpec((_D, 256), lambda i: (0, 0)),
            pl.BlockSpec((1, 256), lambda i: (0, 0)),
        ],
        out_specs=pl.BlockSpec((_BLK, 256), lambda i: (i, 0)),
        out_shape=jax.ShapeDtypeStruct((_N_NODES, 256), jnp.float32),
    )(x, w1a, b1)


def _mlp2_body(x_ref, hx_ref, p0_ref, p1_ref, w1b_ref, w2_ref, b2_ref,
               g_ref, bt_ref, o_ref):
    agg = p0_ref[...] + p1_ref[...]
    h = hx_ref[...] + jnp.dot(agg, w1b_ref[...],
                              preferred_element_type=jnp.float32)
    h = h * lax.logistic(h)
    h = jnp.dot(h, w2_ref[...], preferred_element_type=jnp.float32) + b2_ref[...]
    m = jnp.mean(h, axis=-1, keepdims=True)
    hc = h - m
    v = jnp.mean(hc * hc, axis=-1, keepdims=True)
    o_ref[...] = x_ref[...] + hc * lax.rsqrt(v + 1e-5) * g_ref[...] + bt_ref[...]


def _tc_mlp2(x, hx, partials, w1b, w2, b2, gamma, beta):
    return pl.pallas_call(
        _mlp2_body,
        grid=(_NBLK,),
        in_specs=[
            pl.BlockSpec((_BLK, _D), lambda i: (i, 0)),
            pl.BlockSpec((_BLK, 256), lambda i: (i, 0)),
            pl.BlockSpec((_BLK, _D), lambda i: (i, 0)),
            pl.BlockSpec((_BLK, _D), lambda i: (_NBLK + i, 0)),
            pl.BlockSpec((_D, 256), lambda i: (0, 0)),
            pl.BlockSpec((256, _D), lambda i: (0, 0)),
            pl.BlockSpec((1, _D), lambda i: (0, 0)),
            pl.BlockSpec((1, _D), lambda i: (0, 0)),
            pl.BlockSpec((1, _D), lambda i: (0, 0)),
        ],
        out_specs=pl.BlockSpec((_BLK, _D), lambda i: (i, 0)),
        out_shape=jax.ShapeDtypeStruct((_N_NODES, _D), jnp.float32),
    )(x, hx, partials, partials, w1b, w2, b2, gamma, beta)


def kernel(x, edge_index, edge_attr, W1, b1, W2, b2, gamma, beta):
    j = edge_index[1].astype(jnp.int32)
    partials = _sc_scatter()(j, edge_attr)
    hx = _tc_mlp1(x, W1[:_D], b1.reshape(1, -1))
    return _tc_mlp2(x, hx, partials, W1[_D:], W2, b2.reshape(1, -1),
                    gamma.reshape(1, -1), beta.reshape(1, -1))


# speedup vs baseline: 1.0828x; 1.0828x over previous
import functools

import jax
import jax.numpy as jnp
from jax import lax
from jax.experimental import pallas as pl
from jax.experimental.pallas import tpu as pltpu
from jax.experimental.pallas import tpu_sc as plsc

_N_NODES = 10000
_D = 128
_N_EDGES = 320000
_NC = 2
_NS = 16
_EPW = _N_EDGES // (_NC * _NS)
_CH = 40
_NCHUNK = _EPW // _CH
_NP = 10240
_RPT = _NP // _NS


_NBUF = 5
_NOUT = _NCHUNK // _NBUF


def _sc_body(j_hbm, ea_hbm, out_hbm, idx_v, rows_v, agg_sh, isem, rsem):
    c = lax.axis_index("c")
    s = lax.axis_index("s")
    zv = jnp.zeros((16,), jnp.float32)
    for r in range(_CH):
        for c16 in range(_D // 16):
            rows_v[0, r, pl.ds(c16 * 16, 16)] = zv
    for k in range(_RPT // _CH):
        pltpu.async_copy(rows_v.at[0],
                         agg_sh.at[pl.ds(s * _RPT + k * _CH, _CH)],
                         isem.at[0])
    for k in range(_RPT // _CH):
        pltpu.make_async_copy(rows_v.at[0],
                              agg_sh.at[pl.ds(s * _RPT + k * _CH, _CH)],
                              isem.at[0]).wait()
    plsc.subcore_barrier()
    ebase0 = (c * _NS + s) * _EPW
    jbase0 = _N_EDGES + ebase0

    for b in range(_NBUF):
        off = b * _CH
        pltpu.async_copy(j_hbm.at[pl.ds(jbase0 + off, _CH)], idx_v.at[b],
                         isem.at[b])
        pltpu.async_copy(ea_hbm.at[pl.ds(ebase0 + off, _CH)], rows_v.at[b],
                         rsem.at[b])

    def outer(o, carry):
        for b in range(_NBUF):
            off = (o * _NBUF + b) * _CH
            pltpu.make_async_copy(j_hbm.at[pl.ds(jbase0 + off, _CH)],
                                  idx_v.at[b], isem.at[b]).wait()
            pltpu.make_async_copy(ea_hbm.at[pl.ds(ebase0 + off, _CH)],
                                  rows_v.at[b], rsem.at[b]).wait()
            pltpu.sync_copy(rows_v.at[b], agg_sh.at[idx_v.at[b]], add=True)

            @pl.when(o < _NOUT - 1)
            def _():
                noff = off + _NBUF * _CH
                pltpu.async_copy(j_hbm.at[pl.ds(jbase0 + noff, _CH)],
                                 idx_v.at[b], isem.at[b])
                pltpu.async_copy(ea_hbm.at[pl.ds(ebase0 + noff, _CH)],
                                 rows_v.at[b], rsem.at[b])
        return carry

    lax.fori_loop(0, _NOUT, outer, 0)
    plsc.subcore_barrier()
    dst = out_hbm.at[pl.ds(c * _NP + s * _RPT, _RPT)]
    pltpu.sync_copy(agg_sh.at[pl.ds(s * _RPT, _RPT)], dst)


@functools.cache
def _sc_scatter():
    return pl.kernel(
        _sc_body,
        out_type=jax.ShapeDtypeStruct((_NC * _NP, _D), jnp.float32),
        mesh=plsc.VectorSubcoreMesh(core_axis_name="c", subcore_axis_name="s",
                                    num_cores=_NC, num_subcores=_NS),
        scratch_types=[
            pltpu.VMEM((_NBUF, _CH), jnp.int32),
            pltpu.VMEM((_NBUF, _CH, _D), jnp.float32),
            pltpu.VMEM_SHARED((_NP, _D), jnp.float32),
            pltpu.SemaphoreType.DMA((_NBUF,)),
            pltpu.SemaphoreType.DMA((_NBUF,)),
        ],
    )


_BLK = 1280
_NBLK = _NP // _BLK


def _mlp1_body(x_ref, w1a_ref, b1_ref, hx_ref):
    hx_ref[...] = (jnp.dot(x_ref[...], w1a_ref[...],
                           preferred_element_type=jnp.float32) + b1_ref[...])


def _tc_mlp1(x, w1a, b1):
    return pl.pallas_call(
        _mlp1_body,
        grid=(_NBLK,),
        in_specs=[
            pl.BlockSpec((_BLK, _D), lambda i: (i, 0)),
            pl.BlockSpec((_D, 256), lambda i: (0, 0)),
            pl.BlockSpec((1, 256), lambda i: (0, 0)),
        ],
        out_specs=pl.BlockSpec((_BLK, 256), lambda i: (i, 0)),
        out_shape=jax.ShapeDtypeStruct((_N_NODES, 256), jnp.float32),
    )(x, w1a, b1)


def _mlp2_body(x_ref, hx_ref, p0_ref, p1_ref, w1b_ref, w2_ref, b2_ref,
               g_ref, bt_ref, o_ref):
    agg = p0_ref[...] + p1_ref[...]
    h = hx_ref[...] + jnp.dot(agg, w1b_ref[...],
                              preferred_element_type=jnp.float32)
    h = h * lax.logistic(h)
    h = jnp.dot(h, w2_ref[...], preferred_element_type=jnp.float32) + b2_ref[...]
    m = jnp.mean(h, axis=-1, keepdims=True)
    hc = h - m
    v = jnp.mean(hc * hc, axis=-1, keepdims=True)
    o_ref[...] = x_ref[...] + hc * lax.rsqrt(v + 1e-5) * g_ref[...] + bt_ref[...]


def _tc_mlp2(x, hx, partials, w1b, w2, b2, gamma, beta):
    return pl.pallas_call(
        _mlp2_body,
        grid=(_NBLK,),
        in_specs=[
            pl.BlockSpec((_BLK, _D), lambda i: (i, 0)),
            pl.BlockSpec((_BLK, 256), lambda i: (i, 0)),
            pl.BlockSpec((_BLK, _D), lambda i: (i, 0)),
            pl.BlockSpec((_BLK, _D), lambda i: (_NBLK + i, 0)),
            pl.BlockSpec((_D, 256), lambda i: (0, 0)),
            pl.BlockSpec((256, _D), lambda i: (0, 0)),
            pl.BlockSpec((1, _D), lambda i: (0, 0)),
            pl.BlockSpec((1, _D), lambda i: (0, 0)),
            pl.BlockSpec((1, _D), lambda i: (0, 0)),
        ],
        out_specs=pl.BlockSpec((_BLK, _D), lambda i: (i, 0)),
        out_shape=jax.ShapeDtypeStruct((_N_NODES, _D), jnp.float32),
    )(x, hx, partials, partials, w1b, w2, b2, gamma, beta)


def kernel(x, edge_index, edge_attr, W1, b1, W2, b2, gamma, beta):
    if edge_index.dtype != jnp.int32:
        edge_index = edge_index.astype(jnp.int32)
    ij = edge_index.reshape(-1)
    partials = _sc_scatter()(ij, edge_attr)
    hx = _tc_mlp1(x, W1[:_D], b1.reshape(1, -1))
    return _tc_mlp2(x, hx, partials, W1[_D:], W2, b2.reshape(1, -1),
                    gamma.reshape(1, -1), beta.reshape(1, -1))
